# baseline (speedup 1.0000x reference)
import functools
import math

import jax
import jax.numpy as jnp
from jax import lax
from jax.experimental import pallas as pl
from jax.experimental.pallas import tpu as pltpu
from jax.experimental.pallas import tpu_sc as plsc

D_MODEL = 512
SCALE = math.sqrt(D_MODEL)

NUM_CORES = 2
NUM_SUBCORES = 16
NUM_WORKERS = NUM_CORES * NUM_SUBCORES
LANES = 16

CHUNK = 64
NBUF = 3
VECS_PER_ROW = D_MODEL // LANES


def _make_kernel(n_batch: int, n_seq: int):
    n_rows = n_batch * n_seq
    b_per_w = n_rows // NUM_WORKERS
    w_per_row = n_seq // b_per_w
    n_chunks = b_per_w // CHUNK

    mesh = plsc.VectorSubcoreMesh(
        core_axis_name="c", subcore_axis_name="s", num_cores=NUM_CORES
    )

    @functools.partial(
        pl.kernel,
        mesh=mesh,
        out_type=jax.ShapeDtypeStruct((n_rows, D_MODEL), jnp.float32),
        scratch_types=[
            pltpu.VMEM((b_per_w,), jnp.int32),
            pltpu.VMEM((NBUF * CHUNK, D_MODEL), jnp.float32),
            pltpu.SemaphoreType.DMA((NBUF,)),
            pltpu.SemaphoreType.DMA((NBUF,)),
        ],
    )
    def emb_kernel(idx_hbm, lut_hbm, out_hbm, idx_v, buf, gsems, osems):
        wid = lax.axis_index("s") * NUM_CORES + lax.axis_index("c")
        base = wid * b_per_w
        pltpu.sync_copy(
            idx_hbm.at[wid // w_per_row, pl.ds((wid % w_per_row) * b_per_w, b_per_w)],
            idx_v,
        )

        def gather_copy(c, slot):
            return pltpu.make_async_copy(
                lut_hbm.at[idx_v.at[pl.ds(c * CHUNK, CHUNK)]],
                buf.at[pl.ds(slot * CHUNK, CHUNK)],
                gsems.at[slot],
            )

        def out_copy(c, slot):
            return pltpu.make_async_copy(
                buf.at[pl.ds(slot * CHUNK, CHUNK)],
                out_hbm.at[pl.ds(base + c * CHUNK, CHUNK)],
                osems.at[slot],
            )

        def prime(c, _):
            gather_copy(c, c).start()
            return _

        lax.fori_loop(0, NBUF - 1, prime, None)

        def step(c, _):
            slot = lax.rem(c, NBUF)
            gather_copy(c, slot).wait()

            @plsc.parallel_loop(0, CHUNK, unroll=2)
            def _row(r):
                row = slot * CHUNK + r
                for j in range(VECS_PER_ROW):
                    sl = pl.ds(j * LANES, LANES)
                    buf[row, sl] = buf[row, sl] * SCALE

            out_copy(c, slot).start()
            nxt = c + NBUF - 1

            @pl.when(nxt < n_chunks)
            def _():
                @pl.when(c >= 1)
                def _():
                    out_copy(c - 1, lax.rem(c - 1, NBUF)).wait()

                gather_copy(nxt, lax.rem(nxt, NBUF)).start()

            return _

        lax.fori_loop(0, n_chunks, step, None)

        def drain(c, _):
            out_copy(c, lax.rem(c, NBUF)).wait()
            return _

        lax.fori_loop(max(n_chunks - NBUF + 1, 0), n_chunks, drain, None)

    return emb_kernel


@jax.jit
def kernel(X, lut):
    n_batch, n_seq = X.shape
    out = _make_kernel(n_batch, n_seq)(X.astype(jnp.int32), lut)
    return out.reshape(n_batch, n_seq, D_MODEL)


# speedup vs baseline: 1.0717x; 1.0717x over previous
import functools
import math

import jax
import jax.numpy as jnp
from jax import lax
from jax.experimental import pallas as pl
from jax.experimental.pallas import tpu as pltpu
from jax.experimental.pallas import tpu_sc as plsc

D_MODEL = 512
SCALE = math.sqrt(D_MODEL)

NUM_CORES = 2
NUM_SUBCORES = 16
NUM_WORKERS = NUM_CORES * NUM_SUBCORES
LANES = 16

CHUNK = 32
NBUF = 7
VECS_PER_ROW = D_MODEL // LANES


def _make_kernel(n_batch: int, n_seq: int):
    n_rows = n_batch * n_seq
    b_per_w = n_rows // NUM_WORKERS
    w_per_row = n_seq // b_per_w
    n_chunks = b_per_w // CHUNK

    mesh = plsc.VectorSubcoreMesh(
        core_axis_name="c", subcore_axis_name="s", num_cores=NUM_CORES
    )

    @functools.partial(
        pl.kernel,
        mesh=mesh,
        out_type=jax.ShapeDtypeStruct((n_rows, D_MODEL), jnp.float32),
        scratch_types=[
            pltpu.VMEM((b_per_w,), jnp.int32),
            pltpu.VMEM((NBUF * CHUNK, D_MODEL), jnp.float32),
            pltpu.SemaphoreType.DMA((NBUF,)),
            pltpu.SemaphoreType.DMA((NBUF,)),
        ],
    )
    def emb_kernel(idx_hbm, lut_hbm, out_hbm, idx_v, buf, gsems, osems):
        wid = lax.axis_index("s") * NUM_CORES + lax.axis_index("c")
        base = wid * b_per_w
        pltpu.sync_copy(
            idx_hbm.at[wid // w_per_row, pl.ds((wid % w_per_row) * b_per_w, b_per_w)],
            idx_v,
        )

        def gather_copy(c, slot):
            return pltpu.make_async_copy(
                lut_hbm.at[idx_v.at[pl.ds(c * CHUNK, CHUNK)]],
                buf.at[pl.ds(slot * CHUNK, CHUNK)],
                gsems.at[slot],
            )

        def out_copy(c, slot):
            return pltpu.make_async_copy(
                buf.at[pl.ds(slot * CHUNK, CHUNK)],
                out_hbm.at[pl.ds(base + c * CHUNK, CHUNK)],
                osems.at[slot],
            )

        def prime(c, _):
            gather_copy(c, c).start()
            return _

        lax.fori_loop(0, NBUF - 1, prime, None)

        def step(c, _):
            slot = lax.rem(c, NBUF)
            gather_copy(c, slot).wait()

            @plsc.parallel_loop(0, CHUNK, unroll=2)
            def _row(r):
                row = slot * CHUNK + r
                for j in range(VECS_PER_ROW):
                    sl = pl.ds(j * LANES, LANES)
                    buf[row, sl] = buf[row, sl] * SCALE

            out_copy(c, slot).start()
            nxt = c + NBUF - 1

            @pl.when(nxt < n_chunks)
            def _():
                @pl.when(c >= 1)
                def _():
                    out_copy(c - 1, lax.rem(c - 1, NBUF)).wait()

                gather_copy(nxt, lax.rem(nxt, NBUF)).start()

            return _

        lax.fori_loop(0, n_chunks, step, None)

        def drain(c, _):
            out_copy(c, lax.rem(c, NBUF)).wait()
            return _

        lax.fori_loop(max(n_chunks - NBUF + 1, 0), n_chunks, drain, None)

    return emb_kernel


@jax.jit
def kernel(X, lut):
    n_batch, n_seq = X.shape
    out = _make_kernel(n_batch, n_seq)(X.astype(jnp.int32), lut)
    return out.reshape(n_batch, n_seq, D_MODEL)
